# Initial kernel scaffold; baseline (speedup 1.0000x reference)
import jax
import jax.numpy as jnp
from jax.experimental import pallas as pl


def kernel(x, W_in_w, W_in_b, W_local, W_out_w, W_out_b, neighbor_idx):
    raise NotImplementedError("write your pallas kernel here")


# speedup vs baseline: 13.2622x; 13.2622x over previous
import functools

import jax
import jax.numpy as jnp
from jax.experimental import pallas as pl
from jax.experimental.pallas import tpu as pltpu

_CUBE = 24
_N = _CUBE ** 3
_B = 64
_IN = 512
_OUT = 128
_STEPS = 6
_PAD = 640
_OFFS = tuple(dz * _CUBE * _CUBE + dy * _CUBE + dx
              for dz in (-1, 0, 1) for dy in (-1, 0, 1) for dx in (-1, 0, 1))


def _xproj_body(x_ref, w_ref, b_ref, out_ref, *, g):
    i = pl.program_id(0)
    out_ref[...] = jax.lax.dot_general(
        x_ref[...], w_ref[...], (((1,), (1,)), ((), ())),
        preferred_element_type=jnp.float32) + b_ref[:, pl.ds(i * g, g)]


_CHUNK = 512


def _steps_body(xp_ref, wl_ref, ni_ref, out_ref, hp_ref, h2_ref, we_ref):
    we_ref[...] = jnp.where(ni_ref[...] != _N, wl_ref[...], 0.0)
    hp_ref[...] = jnp.zeros(hp_ref.shape, jnp.float32)

    def chunk(c, carry):
        c0 = pl.multiple_of(c * _CHUNK, 128)
        blk = hp_ref[:, pl.ds(c0, _CHUNK + 2 * _PAD)]
        acc = xp_ref[:, pl.ds(c0, _CHUNK)]
        for k, off in enumerate(_OFFS):
            sh = blk[:, _PAD + off:_PAD + off + _CHUNK]
            acc = acc + sh * we_ref[k:k + 1, pl.ds(c0, _CHUNK)]
        h2_ref[:, pl.ds(c0, _CHUNK)] = jnp.tanh(acc)
        return carry

    def step(_, carry):
        jax.lax.fori_loop(0, _N // _CHUNK, chunk, 0)
        hp_ref[:, _PAD:_PAD + _N] = h2_ref[...]
        return carry

    jax.lax.fori_loop(0, _STEPS, step, 0)
    out_ref[...] = h2_ref[...]


def _outproj_body(h_ref, wo_ref, b_ref, out_ref):
    out_ref[...] = jax.lax.dot_general(
        h_ref[...], wo_ref[...], (((1,), (1,)), ((), ())),
        preferred_element_type=jnp.float32) + b_ref[...]


@jax.jit
def kernel(x, W_in_w, W_in_b, W_local, W_out_w, W_out_b, neighbor_idx):
    wlT = W_local.T
    niT = neighbor_idx.T
    b_in = W_in_b[None, :]
    b_out = W_out_b[None, :]

    grid_t = 12
    g = _N // grid_t
    xp = pl.pallas_call(
        functools.partial(_xproj_body, g=g),
        grid=(grid_t,),
        in_specs=[
            pl.BlockSpec((_B, _IN), lambda i: (0, 0)),
            pl.BlockSpec((g, _IN), lambda i: (i, 0)),
            pl.BlockSpec((1, _N), lambda i: (0, 0)),
        ],
        out_specs=pl.BlockSpec((_B, g), lambda i: (0, i)),
        out_shape=jax.ShapeDtypeStruct((_B, _N), jnp.float32),
    )(x, W_in_w, b_in)

    ht = pl.pallas_call(
        _steps_body,
        scratch_shapes=[pltpu.VMEM((_B, _N + 2 * _PAD), jnp.float32),
                        pltpu.VMEM((_B, _N), jnp.float32),
                        pltpu.VMEM((27, _N), jnp.float32)],
        out_shape=jax.ShapeDtypeStruct((_B, _N), jnp.float32),
    )(xp, wlT, niT)

    out = pl.pallas_call(
        _outproj_body,
        out_shape=jax.ShapeDtypeStruct((_B, _OUT), jnp.float32),
    )(ht, W_out_w, b_out)
    return out


# speedup vs baseline: 23.7311x; 1.7894x over previous
import functools

import jax
import jax.numpy as jnp
from jax.experimental import pallas as pl
from jax.experimental.pallas import tpu as pltpu

_CUBE = 24
_N = _CUBE ** 3
_B = 64
_IN = 512
_OUT = 128
_STEPS = 6

_S = 640
_NP = _CUBE * _S
_HPAD = 768
_UPAD = 640
_UW = _NP + 2 * _UPAD
_DYX = tuple((dy, dx) for dy in (-1, 0, 1) for dx in (-1, 0, 1))

_CA = 256
_CB = 512


def _xproj_body(x_ref, w_ref, b_ref, out_ref, *, g):
    i = pl.program_id(0)
    out_ref[...] = jax.lax.dot_general(
        x_ref[...], w_ref[...], (((1,), (1,)), ((), ())),
        preferred_element_type=jnp.float32) + b_ref[:, pl.ds(i * g, g)]


def _steps_body(xp_ref, we_ref, out_ref, hp_ref, u_ref):
    hp_ref[...] = jnp.zeros(hp_ref.shape, jnp.float32)

    def chunk_a(c, carry):
        j0 = pl.multiple_of(c * _CA, 128)
        blk = hp_ref[:, pl.ds(j0, _CA + 256)]
        for u_i, (dy, dx) in enumerate(_DYX):
            d = dy * _CUBE + dx
            u_ref[u_i, :, pl.ds(j0, _CA)] = blk[:, 128 + d:128 + d + _CA]
        return carry

    def chunk_b(c, carry):
        c0 = pl.multiple_of(c * _CB, 128)
        acc = xp_ref[:, pl.ds(c0, _CB)]
        k = 0
        for dz in (-1, 0, 1):
            base = pl.multiple_of(_UPAD + dz * _S + c0, 128)
            for u_i in range(9):
                sh = u_ref[u_i, :, pl.ds(base, _CB)]
                acc = acc + sh * we_ref[k:k + 1, pl.ds(c0, _CB)]
                k += 1
        hp_ref[:, pl.ds(_HPAD + c0, _CB)] = jnp.tanh(acc)
        return carry

    def step(_, carry):
        jax.lax.fori_loop(0, _UW // _CA, chunk_a, 0)
        jax.lax.fori_loop(0, _NP // _CB, chunk_b, 0)
        return carry

    jax.lax.fori_loop(0, _STEPS, step, 0)

    def chunk_out(c, carry):
        c0 = pl.multiple_of(c * _CB, 128)
        out_ref[:, pl.ds(c0, _CB)] = hp_ref[:, pl.ds(_HPAD + c0, _CB)]
        return carry

    jax.lax.fori_loop(0, _NP // _CB, chunk_out, 0)


def _outproj_body(h_ref, wo_ref, b_ref, out_ref):
    out_ref[...] = jax.lax.dot_general(
        h_ref[...], wo_ref[...], (((1,), (0,)), ((), ())),
        preferred_element_type=jnp.float32) + b_ref[...]


def _pad_slabs(a):
    a3 = a.reshape(_CUBE, _CUBE * _CUBE, -1)
    a3 = jnp.pad(a3, ((0, 0), (0, _S - _CUBE * _CUBE), (0, 0)))
    return a3.reshape(_NP, -1)


@jax.jit
def kernel(x, W_in_w, W_in_b, W_local, W_out_w, W_out_b, neighbor_idx):
    w_eff = jnp.where(neighbor_idx != _N, W_local, 0.0)
    weT = _pad_slabs(w_eff).T
    w_in_p = _pad_slabs(W_in_w)
    b_in_p = _pad_slabs(W_in_b[:, None]).T
    w_out_p = _pad_slabs(W_out_w.T)
    b_out = W_out_b[None, :]

    grid_t = 12
    g = _NP // grid_t
    xp = pl.pallas_call(
        functools.partial(_xproj_body, g=g),
        grid=(grid_t,),
        in_specs=[
            pl.BlockSpec((_B, _IN), lambda i: (0, 0)),
            pl.BlockSpec((g, _IN), lambda i: (i, 0)),
            pl.BlockSpec((1, _NP), lambda i: (0, 0)),
        ],
        out_specs=pl.BlockSpec((_B, g), lambda i: (0, i)),
        out_shape=jax.ShapeDtypeStruct((_B, _NP), jnp.float32),
    )(x, w_in_p, b_in_p)

    ht = pl.pallas_call(
        _steps_body,
        scratch_shapes=[pltpu.VMEM((_B, _NP + 2 * _HPAD), jnp.float32),
                        pltpu.VMEM((9, _B, _UW), jnp.float32)],
        out_shape=jax.ShapeDtypeStruct((_B, _NP), jnp.float32),
    )(xp, weT)

    out = pl.pallas_call(
        _outproj_body,
        out_shape=jax.ShapeDtypeStruct((_B, _OUT), jnp.float32),
    )(ht, w_out_p, b_out)
    return out


# speedup vs baseline: 25.8810x; 1.0906x over previous
import functools

import jax
import jax.numpy as jnp
from jax.experimental import pallas as pl
from jax.experimental.pallas import tpu as pltpu

_CUBE = 24
_N = _CUBE ** 3
_B = 64
_IN = 512
_OUT = 128
_STEPS = 6

_S = 640
_NP = _CUBE * _S
_HPAD = 896
_UPAD = 768
_UW = _NP + 2 * _UPAD
_DYX = tuple((dy, dx) for dy in (-1, 0, 1) for dx in (-1, 0, 1))
_DYX8 = tuple(p for p in _DYX if p != (0, 0))

_CA = 512
_CB = 640


def _xproj_body(x_ref, w_ref, b_ref, out_ref, *, g):
    i = pl.program_id(0)
    out_ref[...] = jax.lax.dot_general(
        x_ref[...], w_ref[...], (((1,), (1,)), ((), ())),
        preferred_element_type=jnp.float32) + b_ref[:, pl.ds(i * g, g)]


def _steps_body(xp_ref, we_ref, out_ref, hpa_ref, hpb_ref, u_ref):
    hpa_ref[...] = jnp.zeros(hpa_ref.shape, jnp.float32)
    hpb_ref[...] = jnp.zeros(hpb_ref.shape, jnp.float32)

    for i in range(_STEPS):
        src_ref, dst_ref = (hpa_ref, hpb_ref) if i % 2 == 0 else (hpb_ref, hpa_ref)

        def chunk_a(c, carry, src_ref=src_ref):
            j0 = pl.multiple_of(c * _CA, 128)
            blk = src_ref[:, pl.ds(j0, _CA + 256)]
            for u_i, (dy, dx) in enumerate(_DYX8):
                d = dy * _CUBE + dx
                u_ref[u_i, :, pl.ds(j0, _CA)] = blk[:, 128 + d:128 + d + _CA]
            return carry

        jax.lax.fori_loop(0, _UW // _CA, chunk_a, 0)

        def chunk_b(c, carry, src_ref=src_ref, dst_ref=dst_ref):
            c0 = pl.multiple_of(c * _CB, 128)
            acc = xp_ref[:, pl.ds(c0, _CB)]
            k = 0
            for dz in (-1, 0, 1):
                for dy, dx in _DYX:
                    if (dy, dx) == (0, 0):
                        base = pl.multiple_of(_HPAD + dz * _S + c0, 128)
                        sh = src_ref[:, pl.ds(base, _CB)]
                    else:
                        base = pl.multiple_of(_UPAD + dz * _S + c0, 128)
                        sh = u_ref[_DYX8.index((dy, dx)), :, pl.ds(base, _CB)]
                    acc = acc + sh * we_ref[k:k + 1, pl.ds(c0, _CB)]
                    k += 1
            dst_ref[:, pl.ds(_HPAD + c0, _CB)] = jnp.tanh(acc)
            return carry

        jax.lax.fori_loop(0, _NP // _CB, chunk_b, 0)

    final_ref = hpa_ref if _STEPS % 2 == 0 else hpb_ref

    def chunk_out(c, carry):
        c0 = pl.multiple_of(c * _CB, 128)
        out_ref[:, pl.ds(c0, _CB)] = final_ref[:, pl.ds(_HPAD + c0, _CB)]
        return carry

    jax.lax.fori_loop(0, _NP // _CB, chunk_out, 0)


def _outproj_body(h_ref, wo_ref, b_ref, out_ref):
    out_ref[...] = jax.lax.dot_general(
        h_ref[...], wo_ref[...], (((1,), (0,)), ((), ())),
        preferred_element_type=jnp.float32) + b_ref[...]


def _pad_slabs(a):
    a3 = a.reshape(_CUBE, _CUBE * _CUBE, -1)
    a3 = jnp.pad(a3, ((0, 0), (0, _S - _CUBE * _CUBE), (0, 0)))
    return a3.reshape(_NP, -1)


@jax.jit
def kernel(x, W_in_w, W_in_b, W_local, W_out_w, W_out_b, neighbor_idx):
    w_eff = jnp.where(neighbor_idx != _N, W_local, 0.0)
    weT = _pad_slabs(w_eff).T
    w_in_p = _pad_slabs(W_in_w)
    b_in_p = _pad_slabs(W_in_b[:, None]).T
    w_out_p = _pad_slabs(W_out_w.T)
    b_out = W_out_b[None, :]

    grid_t = 12
    g = _NP // grid_t
    xp = pl.pallas_call(
        functools.partial(_xproj_body, g=g),
        grid=(grid_t,),
        in_specs=[
            pl.BlockSpec((_B, _IN), lambda i: (0, 0)),
            pl.BlockSpec((g, _IN), lambda i: (i, 0)),
            pl.BlockSpec((1, _NP), lambda i: (0, 0)),
        ],
        out_specs=pl.BlockSpec((_B, g), lambda i: (0, i)),
        out_shape=jax.ShapeDtypeStruct((_B, _NP), jnp.float32),
    )(x, w_in_p, b_in_p)

    ht = pl.pallas_call(
        _steps_body,
        scratch_shapes=[pltpu.VMEM((_B, _NP + 2 * _HPAD), jnp.float32),
                        pltpu.VMEM((_B, _NP + 2 * _HPAD), jnp.float32),
                        pltpu.VMEM((8, _B, _UW), jnp.float32)],
        out_shape=jax.ShapeDtypeStruct((_B, _NP), jnp.float32),
    )(xp, weT)

    out = pl.pallas_call(
        _outproj_body,
        out_shape=jax.ShapeDtypeStruct((_B, _OUT), jnp.float32),
    )(ht, w_out_p, b_out)
    return out


# speedup vs baseline: 26.9477x; 1.0412x over previous
import functools

import jax
import jax.numpy as jnp
from jax.experimental import pallas as pl
from jax.experimental.pallas import tpu as pltpu

_CUBE = 24
_N = _CUBE ** 3
_B = 64
_IN = 512
_OUT = 128
_STEPS = 6

_S = 640
_NP = _CUBE * _S
_HPAD = 768
_UPAD = 640
_UW = _NP + 2 * _UPAD
_DYX = tuple((dy, dx) for dy in (-1, 0, 1) for dx in (-1, 0, 1))
_DYX8 = tuple(p for p in _DYX if p != (0, 0))
_C = 640


def _xproj_body(x_ref, w_ref, b_ref, out_ref, *, g):
    i = pl.program_id(0)
    out_ref[...] = jax.lax.dot_general(
        x_ref[...], w_ref[...], (((1,), (1,)), ((), ())),
        preferred_element_type=jnp.float32) + b_ref[:, pl.ds(i * g, g)]


def _steps_body(xp_ref, we_ref, out_ref, hpa_ref, hpb_ref, u_ref):
    hpa_ref[...] = jnp.zeros(hpa_ref.shape, jnp.float32)
    hpb_ref[...] = jnp.zeros(hpb_ref.shape, jnp.float32)

    for i in range(_STEPS):
        src_ref, dst_ref = (hpa_ref, hpb_ref) if i % 2 == 0 else (hpb_ref, hpa_ref)

        def fused(c, carry, src_ref=src_ref, dst_ref=dst_ref):
            j0 = pl.multiple_of(c * _C, 128)
            blk = src_ref[:, pl.ds(j0, _C + 256)]
            for u_i, (dy, dx) in enumerate(_DYX8):
                d = dy * _CUBE + dx
                u_ref[u_i, :, pl.ds(j0, _C)] = blk[:, 128 + d:128 + d + _C]

            @pl.when(c >= 2)
            def _():
                c0 = pl.multiple_of((c - 2) * _C, 128)
                acc = xp_ref[:, pl.ds(c0, _C)]
                k = 0
                for dz in (-1, 0, 1):
                    for dy, dx in _DYX:
                        if (dy, dx) == (0, 0):
                            base = pl.multiple_of(_HPAD + dz * _S + c0, 128)
                            sh = src_ref[:, pl.ds(base, _C)]
                        else:
                            base = pl.multiple_of(_UPAD + dz * _S + c0, 128)
                            sh = u_ref[_DYX8.index((dy, dx)), :, pl.ds(base, _C)]
                        acc = acc + sh * we_ref[k:k + 1, pl.ds(c0, _C)]
                        k += 1
                dst_ref[:, pl.ds(_HPAD + c0, _C)] = jnp.tanh(acc)
            return carry

        jax.lax.fori_loop(0, _UW // _C, fused, 0)

    final_ref = hpa_ref if _STEPS % 2 == 0 else hpb_ref

    def chunk_out(c, carry):
        c0 = pl.multiple_of(c * _C, 128)
        out_ref[:, pl.ds(c0, _C)] = final_ref[:, pl.ds(_HPAD + c0, _C)]
        return carry

    jax.lax.fori_loop(0, _NP // _C, chunk_out, 0)


def _outproj_body(h_ref, wo_ref, b_ref, out_ref):
    out_ref[...] = jax.lax.dot_general(
        h_ref[...], wo_ref[...], (((1,), (0,)), ((), ())),
        preferred_element_type=jnp.float32) + b_ref[...]


def _pad_slabs(a):
    a3 = a.reshape(_CUBE, _CUBE * _CUBE, -1)
    a3 = jnp.pad(a3, ((0, 0), (0, _S - _CUBE * _CUBE), (0, 0)))
    return a3.reshape(_NP, -1)


@jax.jit
def kernel(x, W_in_w, W_in_b, W_local, W_out_w, W_out_b, neighbor_idx):
    w_eff = jnp.where(neighbor_idx != _N, W_local, 0.0)
    weT = _pad_slabs(w_eff).T
    w_in_p = _pad_slabs(W_in_w)
    b_in_p = _pad_slabs(W_in_b[:, None]).T
    w_out_p = _pad_slabs(W_out_w.T)
    b_out = W_out_b[None, :]

    grid_t = 12
    g = _NP // grid_t
    xp = pl.pallas_call(
        functools.partial(_xproj_body, g=g),
        grid=(grid_t,),
        in_specs=[
            pl.BlockSpec((_B, _IN), lambda i: (0, 0)),
            pl.BlockSpec((g, _IN), lambda i: (i, 0)),
            pl.BlockSpec((1, _NP), lambda i: (0, 0)),
        ],
        out_specs=pl.BlockSpec((_B, g), lambda i: (0, i)),
        out_shape=jax.ShapeDtypeStruct((_B, _NP), jnp.float32),
    )(x, w_in_p, b_in_p)

    ht = pl.pallas_call(
        _steps_body,
        scratch_shapes=[pltpu.VMEM((_B, _NP + 2 * _HPAD), jnp.float32),
                        pltpu.VMEM((_B, _NP + 2 * _HPAD), jnp.float32),
                        pltpu.VMEM((8, _B, _UW), jnp.float32)],
        out_shape=jax.ShapeDtypeStruct((_B, _NP), jnp.float32),
    )(xp, weT)

    out = pl.pallas_call(
        _outproj_body,
        out_shape=jax.ShapeDtypeStruct((_B, _OUT), jnp.float32),
    )(ht, w_out_p, b_out)
    return out
